# Initial kernel scaffold; baseline (speedup 1.0000x reference)
import jax
import jax.numpy as jnp
from jax.experimental import pallas as pl


def kernel(node_feat, rbf, edge_index, W1, b1, W2, b2):
    raise NotImplementedError("write your pallas kernel here")


# speedup vs baseline: 1.8194x; 1.8194x over previous
import jax
import jax.numpy as jnp
from jax import lax
from jax.experimental import pallas as pl
from jax.experimental.pallas import tpu as pltpu, tpu_sc as plsc
import functools

N_NODES = 10000
N_EDGES = 320000
RBF_DIM = 16
DIM = 128

N_CORES = 2
N_SUBCORES = 16
CHUNK = 128
CHUNKS_PER_TILE = 80
E_PER_TILE = CHUNK * CHUNKS_PER_TILE
E_PAD = N_CORES * N_SUBCORES * E_PER_TILE
ACC_ROWS = 10240
DUMMY_DST = N_NODES

TC_BLK = 2048
TC_GRID = E_PAD // TC_BLK


def _softplus(x, beta=0.5, threshold=14.0):
    return jnp.where(beta * x > threshold, x,
                     (1.0 / beta) * jnp.log1p(jnp.exp(beta * x)))


def _edge_mlp_body(rbf_ref, w1_ref, b1_ref, w2_ref, b2_ref, out_ref):
    x = rbf_ref[:]
    h = jnp.dot(x, w1_ref[:], preferred_element_type=jnp.float32) + b1_ref[:]
    h = _softplus(h)
    out_ref[:] = jnp.dot(h, w2_ref[:], preferred_element_type=jnp.float32) + b2_ref[:]


def _edge_mlp(rbf_pad, W1, b1, W2, b2):
    return pl.pallas_call(
        _edge_mlp_body,
        grid=(TC_GRID,),
        in_specs=[
            pl.BlockSpec((TC_BLK, RBF_DIM), lambda g: (g, 0)),
            pl.BlockSpec((RBF_DIM, DIM), lambda g: (0, 0)),
            pl.BlockSpec((1, DIM), lambda g: (0, 0)),
            pl.BlockSpec((DIM, DIM), lambda g: (0, 0)),
            pl.BlockSpec((1, DIM), lambda g: (0, 0)),
        ],
        out_specs=pl.BlockSpec((TC_BLK, DIM), lambda g: (g, 0)),
        out_shape=jax.ShapeDtypeStruct((E_PAD, DIM), jnp.float32),
    )(rbf_pad, W1, b1.reshape(1, DIM), W2, b2.reshape(1, DIM))


def _combine_body(p_ref, o_ref):
    o_ref[:] = p_ref[0] + p_ref[1]


def _combine(partials):
    blk = 1024
    return pl.pallas_call(
        _combine_body,
        grid=(ACC_ROWS // blk,),
        in_specs=[pl.BlockSpec((2, blk, DIM), lambda g: (0, g, 0))],
        out_specs=pl.BlockSpec((blk, DIM), lambda g: (g, 0)),
        out_shape=jax.ShapeDtypeStruct((ACC_ROWS, DIM), jnp.float32),
    )(partials)


@functools.partial(
    pl.kernel,
    out_type=jax.ShapeDtypeStruct((N_CORES, ACC_ROWS, DIM), jnp.float32),
    mesh=plsc.VectorSubcoreMesh(core_axis_name="c", subcore_axis_name="s"),
    scratch_types=[
        pltpu.VMEM((CHUNK,), jnp.int32),
        pltpu.VMEM((CHUNK,), jnp.int32),
        pltpu.VMEM((CHUNK, DIM), jnp.float32),
        pltpu.VMEM((CHUNK, DIM), jnp.float32),
        pltpu.VMEM_SHARED((ACC_ROWS, DIM), jnp.float32),
        pltpu.SemaphoreType.DMA,
    ],
)
def _sc_scatter(node_feat, h, src_h, dst_h, out_h,
                src_v, dst_v, rows_v, h_v, acc, sem):
    cid = lax.axis_index("c")
    sid = lax.axis_index("s")

    zeros = jnp.zeros((16,), jnp.float32)

    def zero_row(i, carry):
        for j in range(DIM // 16):
            rows_v[i, pl.ds(j * 16, 16)] = zeros
        return carry
    lax.fori_loop(0, CHUNK, zero_row, 0)
    for k in range(ACC_ROWS // N_SUBCORES // CHUNK):
        pltpu.sync_copy(rows_v, acc.at[pl.ds(sid * (ACC_ROWS // N_SUBCORES)
                                             + k * CHUNK, CHUNK)])
    plsc.subcore_barrier()

    def chunk_body(g, carry):
        base = (cid * N_SUBCORES + sid) * E_PER_TILE + g * CHUNK
        pltpu.sync_copy(src_h.at[pl.ds(base, CHUNK)], src_v)
        pltpu.sync_copy(dst_h.at[pl.ds(base, CHUNK)], dst_v)
        pltpu.async_copy(node_feat.at[src_v], rows_v, sem).wait()
        pltpu.sync_copy(h.at[pl.ds(base, CHUNK)], h_v)

        def mul_row(i, c2):
            for j in range(DIM // 16):
                s = pl.ds(j * 16, 16)
                rows_v[i, s] = rows_v[i, s] * h_v[i, s]
            return c2
        lax.fori_loop(0, CHUNK, mul_row, 0)

        pltpu.sync_copy(rows_v, acc.at[dst_v], add=True)
        return carry
    lax.fori_loop(0, CHUNKS_PER_TILE, chunk_body, 0)

    plsc.subcore_barrier()

    rows_per_tile = ACC_ROWS // N_SUBCORES
    for k in range(rows_per_tile // CHUNK):
        r = sid * rows_per_tile + k * CHUNK
        pltpu.sync_copy(acc.at[pl.ds(r, CHUNK)], rows_v)
        pltpu.sync_copy(rows_v, out_h.at[cid].at[pl.ds(r, CHUNK)])


def kernel(node_feat, rbf, edge_index, W1, b1, W2, b2):
    pad = E_PAD - N_EDGES
    src = jnp.concatenate(
        [edge_index[0].astype(jnp.int32), jnp.zeros((pad,), jnp.int32)])
    dst = jnp.concatenate(
        [edge_index[1].astype(jnp.int32),
         jnp.full((pad,), DUMMY_DST, jnp.int32)])
    rbf_pad = jnp.concatenate(
        [rbf, jnp.zeros((pad, RBF_DIM), jnp.float32)])

    h = _edge_mlp(rbf_pad, W1, b1, W2, b2)
    partials = _sc_scatter(node_feat, h, src, dst)
    return _combine(partials)[:N_NODES]


# speedup vs baseline: 2.4148x; 1.3272x over previous
import jax
import jax.numpy as jnp
from jax import lax
from jax.experimental import pallas as pl
from jax.experimental.pallas import tpu as pltpu, tpu_sc as plsc
import functools

N_NODES = 10000
N_EDGES = 320000
RBF_DIM = 16
DIM = 128

N_CORES = 2
N_SUBCORES = 16
CHUNK = 64
CHUNKS_PER_TILE = 160
E_PER_TILE = CHUNK * CHUNKS_PER_TILE
E_PAD = N_CORES * N_SUBCORES * E_PER_TILE
ACC_ROWS = 10112
DUMMY_DST = N_NODES

TC_BLK = 2048
TC_GRID = E_PAD // TC_BLK


def _softplus(x, beta=0.5, threshold=14.0):
    return jnp.where(beta * x > threshold, x,
                     (1.0 / beta) * jnp.log1p(jnp.exp(beta * x)))


def _edge_mlp_body(rbf_ref, w1_ref, b1_ref, w2_ref, b2_ref, out_ref):
    x = rbf_ref[:]
    h = jnp.dot(x, w1_ref[:], preferred_element_type=jnp.float32) + b1_ref[:]
    h = _softplus(h)
    out_ref[:] = jnp.dot(h, w2_ref[:], preferred_element_type=jnp.float32) + b2_ref[:]


def _edge_mlp(rbf_pad, W1, b1, W2, b2):
    return pl.pallas_call(
        _edge_mlp_body,
        grid=(TC_GRID,),
        in_specs=[
            pl.BlockSpec((TC_BLK, RBF_DIM), lambda g: (g, 0)),
            pl.BlockSpec((RBF_DIM, DIM), lambda g: (0, 0)),
            pl.BlockSpec((1, DIM), lambda g: (0, 0)),
            pl.BlockSpec((DIM, DIM), lambda g: (0, 0)),
            pl.BlockSpec((1, DIM), lambda g: (0, 0)),
        ],
        out_specs=pl.BlockSpec((TC_BLK, DIM), lambda g: (g, 0)),
        out_shape=jax.ShapeDtypeStruct((E_PAD, DIM), jnp.float32),
    )(rbf_pad, W1, b1.reshape(1, DIM), W2, b2.reshape(1, DIM))


def _combine_body(p_ref, o_ref):
    o_ref[:] = p_ref[0] + p_ref[1]


def _combine(partials):
    blk = 632
    return pl.pallas_call(
        _combine_body,
        grid=(ACC_ROWS // blk,),
        in_specs=[pl.BlockSpec((2, blk, DIM), lambda g: (0, g, 0))],
        out_specs=pl.BlockSpec((blk, DIM), lambda g: (g, 0)),
        out_shape=jax.ShapeDtypeStruct((ACC_ROWS, DIM), jnp.float32),
    )(partials)


@functools.partial(
    pl.kernel,
    out_type=jax.ShapeDtypeStruct((N_CORES, ACC_ROWS, DIM), jnp.float32),
    mesh=plsc.VectorSubcoreMesh(core_axis_name="c", subcore_axis_name="s"),
    scratch_types=[
        [pltpu.VMEM((2, CHUNK), jnp.int32)] * 4,
        [pltpu.VMEM((CHUNK, DIM), jnp.float32)] * 2,
        [pltpu.VMEM((CHUNK, DIM), jnp.float32)] * 2,
        [pltpu.VMEM((CHUNK, DIM), jnp.float32)] * 2,
        pltpu.VMEM_SHARED((ACC_ROWS, DIM), jnp.float32),
        [pltpu.SemaphoreType.DMA] * 4,
        [pltpu.SemaphoreType.DMA] * 2,
        [pltpu.SemaphoreType.DMA] * 2,
        [pltpu.SemaphoreType.DMA] * 2,
    ],
)
def _sc_scatter(node_feat, h, idx2_h, out_h,
                idx, rows, hbuf, msg, acc, isem, gsem, hsem, ssem):
    cid = lax.axis_index("c")
    sid = lax.axis_index("s")
    wid = cid * N_SUBCORES + sid
    crow = wid * CHUNKS_PER_TILE

    zeros = jnp.zeros((16,), jnp.float32)

    def zero_row(i, carry):
        for j in range(DIM // 16):
            rows[0][i, pl.ds(j * 16, 16)] = zeros
        return carry
    lax.fori_loop(0, CHUNK, zero_row, 0)
    rows_per_tile = ACC_ROWS // N_SUBCORES
    for k in range(10):
        r = sid * rows_per_tile + k * CHUNK
        n = CHUNK if k < 9 else rows_per_tile - 9 * CHUNK
        pltpu.sync_copy(rows[0].at[pl.ds(0, n)], acc.at[pl.ds(r, n)])
    plsc.subcore_barrier()

    def start_idx(g, q):
        pltpu.async_copy(idx2_h.at[crow + g], idx[q], isem[q])

    def start_in(g, b, q):
        pltpu.async_copy(node_feat.at[idx[q].at[0]], rows[b], gsem[b])
        pltpu.async_copy(h.at[pl.ds((crow + g) * CHUNK, CHUNK)], hbuf[b],
                         hsem[b])

    start_idx(0, 0)
    start_idx(1, 1)
    pltpu.make_async_copy(idx2_h.at[crow], idx[0], isem[0]).wait()
    start_in(0, 0, 0)

    n_outer = CHUNKS_PER_TILE // 4
    def outer(t, carry):
        for p in range(4):
            g = 4 * t + p
            b, nb, q = p & 1, 1 - (p & 1), p
            nq, q2 = (p + 1) % 4, (p + 2) % 4
            if p == 3:
                @pl.when(t < n_outer - 1)
                def _():
                    pltpu.make_async_copy(idx2_h.at[crow], idx[nq],
                                          isem[nq]).wait()
                    start_in(g + 1, nb, nq)
            else:
                pltpu.make_async_copy(idx2_h.at[crow], idx[nq],
                                      isem[nq]).wait()
                start_in(g + 1, nb, nq)
            pltpu.make_async_copy(node_feat.at[idx[q].at[0]], rows[b],
                                  gsem[b]).wait()
            pltpu.make_async_copy(h.at[pl.ds(0, CHUNK)], hbuf[b],
                                  hsem[b]).wait()
            if p < 2:
                @pl.when(t > 0)
                def _():
                    pltpu.make_async_copy(msg[b], acc.at[idx[q].at[1]],
                                          ssem[b]).wait()
            else:
                pltpu.make_async_copy(msg[b], acc.at[idx[q].at[1]],
                                      ssem[b]).wait()
            if p >= 2:
                @pl.when(t < n_outer - 1)
                def _():
                    start_idx(g + 2, q2)
            else:
                start_idx(g + 2, q2)

            def mul_row(i, c2):
                for j in range(DIM // 16):
                    s = pl.ds(j * 16, 16)
                    msg[b][i, s] = rows[b][i, s] * hbuf[b][i, s]
                return c2
            lax.fori_loop(0, CHUNK, mul_row, 0)

            pltpu.async_copy(msg[b], acc.at[idx[q].at[1]], ssem[b], add=True)
        return carry
    lax.fori_loop(0, n_outer, outer, 0)

    for b in range(2):
        pltpu.make_async_copy(msg[b], acc.at[idx[b].at[1]], ssem[b]).wait()

    plsc.subcore_barrier()

    for k in range(10):
        r = sid * rows_per_tile + k * CHUNK
        n = CHUNK if k < 9 else rows_per_tile - 9 * CHUNK
        pltpu.sync_copy(acc.at[pl.ds(r, n)], rows[0].at[pl.ds(0, n)])
        pltpu.sync_copy(rows[0].at[pl.ds(0, n)], out_h.at[cid].at[pl.ds(r, n)])


def kernel(node_feat, rbf, edge_index, W1, b1, W2, b2):
    pad = E_PAD - N_EDGES
    src = jnp.concatenate(
        [edge_index[0].astype(jnp.int32), jnp.zeros((pad,), jnp.int32)])
    dst = jnp.concatenate(
        [edge_index[1].astype(jnp.int32),
         jnp.full((pad,), DUMMY_DST, jnp.int32)])
    rbf_pad = jnp.concatenate(
        [rbf, jnp.zeros((pad, RBF_DIM), jnp.float32)])

    idx2 = jnp.stack([src.reshape(E_PAD // CHUNK, CHUNK),
                      dst.reshape(E_PAD // CHUNK, CHUNK)], axis=1)
    h = _edge_mlp(rbf_pad, W1, b1, W2, b2)
    partials = _sc_scatter(node_feat, h, idx2)
    return _combine(partials)[:N_NODES]
